# Initial kernel scaffold; baseline (speedup 1.0000x reference)
import jax
import jax.numpy as jnp
from jax.experimental import pallas as pl


def kernel(node_positions, primitives, init):
    raise NotImplementedError("write your pallas kernel here")


# speedup vs baseline: 7.8246x; 7.8246x over previous
import functools

import jax
import jax.numpy as jnp
from jax import lax
from jax.experimental import pallas as pl
from jax.experimental.pallas import tpu as pltpu
from jax.experimental.pallas import tpu_sc as plsc

DIM = 128
N_ROWS = 1024
N_POS = 327680
NC, NS = 2, 16
NW = NC * NS
PER_W = N_POS // NW
CHUNK = 128
N_CHUNKS = PER_W // CHUNK


def _table_body(prim_ref, init_ref, out_ref):
    p0 = prim_ref[0]
    p1 = prim_ref[1]
    out_ref[0:1, :] = init_ref[...]
    for k in range(1, 10):
        h = 1 << (k - 1)
        prev = out_ref[pl.ds(h - 1, h), :]
        out_ref[pl.ds(2 * h - 1, h), :] = lax.dot_general(
            prev, p0, (((1,), (1,)), ((), ())),
            preferred_element_type=jnp.float32)
        out_ref[pl.ds(3 * h - 1, h), :] = lax.dot_general(
            prev, p1, (((1,), (1,)), ((), ())),
            preferred_element_type=jnp.float32)
    out_ref[pl.ds(N_ROWS - 1, 1), :] = lax.dot_general(
        out_ref[pl.ds(511, 1), :], p0, (((1,), (1,)), ((), ())),
        preferred_element_type=jnp.float32)


_build_table = pl.pallas_call(
    _table_body,
    out_shape=jax.ShapeDtypeStruct((N_ROWS, DIM), jnp.float32),
)

@functools.cache
def _make_gather():
    mesh = plsc.VectorSubcoreMesh(core_axis_name="c", subcore_axis_name="s")

    @functools.partial(
        pl.kernel,
        mesh=mesh,
        out_type=jax.ShapeDtypeStruct((N_POS, DIM), jnp.float32),
        scratch_types=[
            pltpu.VMEM((PER_W,), jnp.int32),
            pltpu.VMEM((CHUNK, DIM), jnp.float32),
            pltpu.VMEM((CHUNK, DIM), jnp.float32),
            pltpu.SemaphoreType.DMA,
            pltpu.SemaphoreType.DMA,
        ],
    )
    def _gather(table_hbm, idx_hbm, out_hbm, idx_v, rows0, rows1, sem0, sem1):
        wid = lax.axis_index("s") * NC + lax.axis_index("c")
        base = wid * PER_W
        pltpu.sync_copy(idx_hbm.at[pl.ds(base, PER_W)], idx_v)

        def body(j, carry):
            pltpu.async_copy(
                table_hbm.at[idx_v.at[pl.ds(j * CHUNK, CHUNK)]], rows0, sem0
            ).wait()
            pltpu.sync_copy(rows0, out_hbm.at[pl.ds(base + j * CHUNK, CHUNK)])
            return carry

        lax.fori_loop(0, N_CHUNKS, body, 0)

    return _gather


def kernel(node_positions, primitives, init):
    table = _build_table(primitives, init.reshape(1, DIM))
    idx = node_positions - 1
    return _make_gather()(table, idx)


# speedup vs baseline: 8.5826x; 1.0969x over previous
import functools

import jax
import jax.numpy as jnp
from jax import lax
from jax.experimental import pallas as pl
from jax.experimental.pallas import tpu as pltpu
from jax.experimental.pallas import tpu_sc as plsc

DIM = 128
N_ROWS = 1024
N_POS = 327680
NC, NS = 2, 16
NW = NC * NS
PER_W = N_POS // NW
CHUNK = 128
N_CHUNKS = PER_W // CHUNK


def _rowsxpt(rows, p):
    return lax.dot_general(
        rows, p, (((1,), (1,)), ((), ())),
        precision=lax.Precision.HIGHEST,
        preferred_element_type=jnp.float32)


def _table_body(prim_ref, init_ref, out_ref):
    p0 = prim_ref[0]
    p1 = prim_ref[1]
    out_ref[0:1, :] = init_ref[...]
    for k in range(1, 10):
        h = 1 << (k - 1)
        prev = out_ref[pl.ds(h - 1, h), :]
        out_ref[pl.ds(2 * h - 1, h), :] = _rowsxpt(prev, p0)
        out_ref[pl.ds(3 * h - 1, h), :] = _rowsxpt(prev, p1)
    out_ref[pl.ds(N_ROWS - 1, 1), :] = _rowsxpt(out_ref[pl.ds(511, 1), :], p0)


_build_table = pl.pallas_call(
    _table_body,
    out_shape=jax.ShapeDtypeStruct((N_ROWS, DIM), jnp.float32),
)

@functools.cache
def _make_gather():
    mesh = plsc.VectorSubcoreMesh(core_axis_name="c", subcore_axis_name="s")

    @functools.partial(
        pl.kernel,
        mesh=mesh,
        out_type=jax.ShapeDtypeStruct((N_POS, DIM), jnp.float32),
        scratch_types=[
            pltpu.VMEM((PER_W,), jnp.int32),
            pltpu.VMEM((CHUNK, DIM), jnp.float32),
            pltpu.VMEM((CHUNK, DIM), jnp.float32),
            pltpu.SemaphoreType.DMA,
            pltpu.SemaphoreType.DMA,
        ],
    )
    def _gather(table_hbm, idx_hbm, out_hbm, idx_v, rows0, rows1, sem0, sem1):
        wid = lax.axis_index("s") * NC + lax.axis_index("c")
        base = wid * PER_W
        pltpu.sync_copy(idx_hbm.at[pl.ds(base, PER_W)], idx_v)

        bufs = (rows0, rows1)
        sems = (sem0, sem1)

        def fire(j, b):
            pltpu.async_copy(
                table_hbm.at[idx_v.at[pl.ds(j * CHUNK, CHUNK)]], bufs[b], sems[b])

        fire(0, 0)

        def body(j2, carry):
            for b in range(2):
                j = j2 * 2 + b

                @pl.when(j + 1 < N_CHUNKS)
                def _():
                    fire(j + 1, 1 - b)

                pltpu.make_async_copy(
                    table_hbm.at[pl.ds(0, CHUNK)], bufs[b], sems[b]).wait()
                pltpu.sync_copy(
                    bufs[b], out_hbm.at[pl.ds(base + j * CHUNK, CHUNK)])
            return carry

        lax.fori_loop(0, N_CHUNKS // 2, body, 0)

    return _gather


def kernel(node_positions, primitives, init):
    table = _build_table(primitives, init.reshape(1, DIM))
    idx = node_positions - 1
    return _make_gather()(table, idx)


# speedup vs baseline: 19.5894x; 2.2825x over previous
import functools

import jax
import jax.numpy as jnp
from jax import lax
from jax.experimental import pallas as pl
from jax.experimental.pallas import tpu as pltpu
from jax.experimental.pallas import tpu_sc as plsc

DIM = 128
N_ROWS = 1024
N_POS = 327680
NC, NS = 2, 16
NW = NC * NS
PER_W = N_POS // NW
CHUNK = 128
N_CHUNKS = PER_W // CHUNK


def _rowsxpt(rows, p):
    return lax.dot_general(
        rows, p, (((1,), (1,)), ((), ())),
        precision=lax.Precision.HIGHEST,
        preferred_element_type=jnp.float32)


def _table_body(prim_ref, init_ref, out_ref):
    p0 = prim_ref[0]
    p1 = prim_ref[1]
    out_ref[0:1, :] = init_ref[...]
    for k in range(1, 10):
        h = 1 << (k - 1)
        prev = out_ref[pl.ds(h - 1, h), :]
        out_ref[pl.ds(2 * h - 1, h), :] = _rowsxpt(prev, p0)
        out_ref[pl.ds(3 * h - 1, h), :] = _rowsxpt(prev, p1)
    out_ref[pl.ds(N_ROWS - 1, 1), :] = _rowsxpt(out_ref[pl.ds(511, 1), :], p0)


_build_table = pl.pallas_call(
    _table_body,
    out_shape=jax.ShapeDtypeStruct((N_ROWS, DIM), jnp.float32),
)

@functools.cache
def _make_gather():
    mesh = plsc.VectorSubcoreMesh(core_axis_name="c", subcore_axis_name="s")

    @functools.partial(
        pl.kernel,
        mesh=mesh,
        out_type=jax.ShapeDtypeStruct((N_POS, DIM), jnp.float32),
        scratch_types=[
            pltpu.VMEM((PER_W,), jnp.int32),
            pltpu.VMEM((CHUNK, DIM), jnp.float32),
            pltpu.VMEM((CHUNK, DIM), jnp.float32),
            pltpu.VMEM_SHARED((N_ROWS, DIM), jnp.float32),
            pltpu.SemaphoreType.DMA,
            pltpu.SemaphoreType.DMA,
        ],
    )
    def _gather(table_hbm, idx_hbm, out_hbm, idx_v, rows0, rows1, tab_sh,
                sem0, sem1):
        sid = lax.axis_index("s")
        wid = sid * NC + lax.axis_index("c")
        base = wid * PER_W

        @pl.when(sid == 0)
        def _():
            pltpu.sync_copy(table_hbm, tab_sh)

        pltpu.sync_copy(idx_hbm.at[pl.ds(base, PER_W)], idx_v)
        plsc.subcore_barrier()

        bufs = (rows0, rows1)
        sems = (sem0, sem1)

        def fire(j, b):
            pltpu.async_copy(
                tab_sh.at[idx_v.at[pl.ds(j * CHUNK, CHUNK)]], bufs[b], sems[b])

        fire(0, 0)

        def body(j2, carry):
            for b in range(2):
                j = j2 * 2 + b

                @pl.when(j + 1 < N_CHUNKS)
                def _():
                    fire(j + 1, 1 - b)

                pltpu.make_async_copy(
                    table_hbm.at[pl.ds(0, CHUNK)], bufs[b], sems[b]).wait()
                pltpu.sync_copy(
                    bufs[b], out_hbm.at[pl.ds(base + j * CHUNK, CHUNK)])
            return carry

        lax.fori_loop(0, N_CHUNKS // 2, body, 0)

    return _gather


def kernel(node_positions, primitives, init):
    table = _build_table(primitives, init.reshape(1, DIM))
    idx = node_positions - 1
    return _make_gather()(table, idx)


# speedup vs baseline: 19.7172x; 1.0065x over previous
import functools

import jax
import jax.numpy as jnp
from jax import lax
from jax.experimental import pallas as pl
from jax.experimental.pallas import tpu as pltpu
from jax.experimental.pallas import tpu_sc as plsc

DIM = 128
N_ROWS = 1024
N_POS = 327680
NC, NS = 2, 16
NW = NC * NS
PER_W = N_POS // NW
CHUNK = 128
N_CHUNKS = PER_W // CHUNK
NBUF = 5
LOOKAHEAD = 2


def _rowsxpt(rows, p):
    return lax.dot_general(
        rows, p, (((1,), (1,)), ((), ())),
        precision=lax.Precision.HIGHEST,
        preferred_element_type=jnp.float32)


def _table_body(prim_ref, init_ref, out_ref):
    p0 = prim_ref[0]
    p1 = prim_ref[1]
    out_ref[0:1, :] = init_ref[...]
    for k in range(1, 10):
        h = 1 << (k - 1)
        prev = out_ref[pl.ds(h - 1, h), :]
        out_ref[pl.ds(2 * h - 1, h), :] = _rowsxpt(prev, p0)
        out_ref[pl.ds(3 * h - 1, h), :] = _rowsxpt(prev, p1)
    out_ref[pl.ds(N_ROWS - 1, 1), :] = _rowsxpt(out_ref[pl.ds(511, 1), :], p0)


_build_table = pl.pallas_call(
    _table_body,
    out_shape=jax.ShapeDtypeStruct((N_ROWS, DIM), jnp.float32),
)

@functools.cache
def _make_gather():
    mesh = plsc.VectorSubcoreMesh(core_axis_name="c", subcore_axis_name="s")

    @functools.partial(
        pl.kernel,
        mesh=mesh,
        out_type=jax.ShapeDtypeStruct((N_POS, DIM), jnp.float32),
        scratch_types=[
            pltpu.VMEM((PER_W,), jnp.int32),
            pltpu.VMEM((NBUF, CHUNK, DIM), jnp.float32),
            pltpu.VMEM_SHARED((N_ROWS, DIM), jnp.float32),
        ] + [pltpu.SemaphoreType.DMA] * (2 * NBUF),
    )
    def _gather(table_hbm, idx_hbm, out_hbm, idx_v, rows, tab_sh, *sems):
        gsem = sems[:NBUF]
        wsem = sems[NBUF:]
        sid = lax.axis_index("s")
        wid = sid * NC + lax.axis_index("c")
        base = wid * PER_W

        @pl.when(sid == 0)
        def _():
            pltpu.sync_copy(table_hbm, tab_sh)

        pltpu.sync_copy(idx_hbm.at[pl.ds(base, PER_W)], idx_v)
        plsc.subcore_barrier()

        def fire_gather(j, s):
            pltpu.async_copy(
                tab_sh.at[idx_v.at[pl.ds(j * CHUNK, CHUNK)]], rows.at[s],
                gsem[s])

        def wait_gather(s):
            pltpu.make_async_copy(
                table_hbm.at[pl.ds(0, CHUNK)], rows.at[s], gsem[s]).wait()

        def fire_write(j, s):
            pltpu.async_copy(
                rows.at[s], out_hbm.at[pl.ds(base + j * CHUNK, CHUNK)],
                wsem[s])

        def wait_write(s):
            pltpu.make_async_copy(
                rows.at[s], out_hbm.at[pl.ds(base, CHUNK)], wsem[s]).wait()

        for j in range(LOOKAHEAD):
            fire_gather(j, j)

        def body(jo, carry):
            for u in range(NBUF):
                j = jo * NBUF + u
                wait_gather(u)
                fire_write(j, u)

                @pl.when(j + LOOKAHEAD < N_CHUNKS)
                def _():
                    ns = (u + LOOKAHEAD) % NBUF

                    @pl.when(j - (NBUF - LOOKAHEAD) >= 0)
                    def _():
                        wait_write(ns)

                    fire_gather(j + LOOKAHEAD, ns)
            return carry

        lax.fori_loop(0, N_CHUNKS // NBUF, body, 0)
        for s in range(NBUF):
            wait_write(s)

    return _gather


def kernel(node_positions, primitives, init):
    table = _build_table(primitives, init.reshape(1, DIM))
    idx = node_positions - 1
    return _make_gather()(table, idx)


# speedup vs baseline: 19.7885x; 1.0036x over previous
import functools

import jax
import jax.numpy as jnp
from jax import lax
from jax.experimental import pallas as pl
from jax.experimental.pallas import tpu as pltpu
from jax.experimental.pallas import tpu_sc as plsc

DIM = 128
N_ROWS = 1024
N_POS = 327680
NC, NS = 2, 16
NW = NC * NS
PER_W = N_POS // NW
CHUNK = 256
N_CHUNKS = PER_W // CHUNK
NBUF = 2
LOOKAHEAD = 1


def _rowsxpt(rows, p):
    return lax.dot_general(
        rows, p, (((1,), (1,)), ((), ())),
        preferred_element_type=jnp.float32)


def _table_body(prim_ref, init_ref, out_ref):
    p0 = prim_ref[0]
    p1 = prim_ref[1]
    out_ref[0:1, :] = init_ref[...]
    for k in range(1, 10):
        h = 1 << (k - 1)
        prev = out_ref[pl.ds(h - 1, h), :]
        out_ref[pl.ds(2 * h - 1, h), :] = _rowsxpt(prev, p0)
        out_ref[pl.ds(3 * h - 1, h), :] = _rowsxpt(prev, p1)
    out_ref[pl.ds(N_ROWS - 1, 1), :] = _rowsxpt(out_ref[pl.ds(511, 1), :], p0)


_build_table = pl.pallas_call(
    _table_body,
    out_shape=jax.ShapeDtypeStruct((N_ROWS, DIM), jnp.float32),
)

@functools.cache
def _make_gather():
    mesh = plsc.VectorSubcoreMesh(core_axis_name="c", subcore_axis_name="s")

    @functools.partial(
        pl.kernel,
        mesh=mesh,
        out_type=jax.ShapeDtypeStruct((N_POS, DIM), jnp.float32),
        scratch_types=[
            pltpu.VMEM((PER_W,), jnp.int32),
            pltpu.VMEM((NBUF, CHUNK, DIM), jnp.float32),
            pltpu.VMEM_SHARED((N_ROWS, DIM), jnp.float32),
        ] + [pltpu.SemaphoreType.DMA] * (2 * NBUF),
    )
    def _gather(table_hbm, idx_hbm, out_hbm, idx_v, rows, tab_sh, *sems):
        gsem = sems[:NBUF]
        wsem = sems[NBUF:]
        sid = lax.axis_index("s")
        wid = sid * NC + lax.axis_index("c")
        base = wid * PER_W

        @pl.when(sid == 0)
        def _():
            pltpu.sync_copy(table_hbm, tab_sh)

        pltpu.sync_copy(idx_hbm.at[pl.ds(base, PER_W)], idx_v)
        plsc.subcore_barrier()

        def fire_gather(j, s):
            pltpu.async_copy(
                tab_sh.at[idx_v.at[pl.ds(j * CHUNK, CHUNK)]], rows.at[s],
                gsem[s])

        def wait_gather(s):
            pltpu.make_async_copy(
                table_hbm.at[pl.ds(0, CHUNK)], rows.at[s], gsem[s]).wait()

        def fire_write(j, s):
            pltpu.async_copy(
                rows.at[s], out_hbm.at[pl.ds(base + j * CHUNK, CHUNK)],
                wsem[s])

        def wait_write(s):
            pltpu.make_async_copy(
                rows.at[s], out_hbm.at[pl.ds(base, CHUNK)], wsem[s]).wait()

        for j in range(LOOKAHEAD):
            fire_gather(j, j)

        def body(jo, carry):
            for u in range(NBUF):
                j = jo * NBUF + u
                wait_gather(u)
                fire_write(j, u)

                @pl.when(j + LOOKAHEAD < N_CHUNKS)
                def _():
                    ns = (u + LOOKAHEAD) % NBUF

                    @pl.when(j - (NBUF - LOOKAHEAD) >= 0)
                    def _():
                        wait_write(ns)

                    fire_gather(j + LOOKAHEAD, ns)
            return carry

        lax.fori_loop(0, N_CHUNKS // NBUF, body, 0)
        for s in range(NBUF):
            wait_write(s)

    return _gather


def kernel(node_positions, primitives, init):
    table = _build_table(primitives, init.reshape(1, DIM))
    idx = node_positions - 1
    return _make_gather()(table, idx)
